# Initial kernel scaffold; baseline (speedup 1.0000x reference)
import jax
import jax.numpy as jnp
from jax.experimental import pallas as pl


def kernel(x, expert_weights, token_indices, w1, w2, w3):
    raise NotImplementedError("write your pallas kernel here")


# speedup vs baseline: 1.7820x; 1.7820x over previous
import functools

import jax
import jax.numpy as jnp
from jax.experimental import pallas as pl
from jax.experimental.pallas import tpu as pltpu


def _moe_kernel(idx_ref, ew_ref, x_ref, w1_ref, w2_ref, w3_ref, out_ref):
    e = pl.program_id(0)

    @pl.when(e == 0)
    def _init():
        out_ref[...] = jnp.zeros_like(out_ref)

    idx = idx_ref[0, 0, :]
    ew = ew_ref[0, 0, :]
    c = idx.shape[0]
    s = x_ref.shape[0]

    iota = jax.lax.broadcasted_iota(jnp.int32, (c, s), 1)
    onehot = (iota == idx[:, None]).astype(jnp.float32)
    inp = jnp.dot(onehot, x_ref[...], preferred_element_type=jnp.float32)

    gate = jnp.dot(inp, w1_ref[0], preferred_element_type=jnp.float32)
    value = jnp.dot(inp, w2_ref[0], preferred_element_type=jnp.float32)
    hidden = (gate * jax.nn.sigmoid(gate)) * value
    out = jnp.dot(hidden, w3_ref[0], preferred_element_type=jnp.float32)

    weighted = out * ew[:, None]
    out_ref[...] += jnp.dot(onehot.T, weighted, preferred_element_type=jnp.float32)


@functools.partial(jax.jit, static_argnames=("interpret",))
def _run(x, expert_weights, token_indices, w1, w2, w3, interpret=False):
    B, S, D = x.shape
    E, _, F = w1.shape
    C = token_indices.shape[2]

    idx = token_indices.astype(jnp.int32).reshape(E, 1, C)
    ew = expert_weights.astype(jnp.float32).reshape(E, 1, C)
    x2 = x.reshape(S, D)

    out = pl.pallas_call(
        _moe_kernel,
        grid=(E,),
        in_specs=[
            pl.BlockSpec((1, 1, C), lambda e: (e, 0, 0)),
            pl.BlockSpec((1, 1, C), lambda e: (e, 0, 0)),
            pl.BlockSpec((S, D), lambda e: (0, 0)),
            pl.BlockSpec((1, D, F), lambda e: (e, 0, 0)),
            pl.BlockSpec((1, D, F), lambda e: (e, 0, 0)),
            pl.BlockSpec((1, F, D), lambda e: (e, 0, 0)),
        ],
        out_specs=pl.BlockSpec((S, D), lambda e: (0, 0)),
        out_shape=jax.ShapeDtypeStruct((S, D), jnp.float32),
        interpret=interpret,
    )(idx, ew, x2, w1, w2, w3)
    return out.reshape(B, S, D)


def kernel(x, expert_weights, token_indices, w1, w2, w3):
    return _run(x, expert_weights, token_indices, w1, w2, w3)
